# baseline (speedup 1.0000x reference)
import functools

import jax
import jax.numpy as jnp
from jax import lax
from jax.experimental import pallas as pl
from jax.experimental.pallas import tpu as pltpu
from jax.experimental.pallas import tpu_sc as plsc

D = 64
SEQ = 200
NSLICE = D // 16
N_CORES = 2
N_SUBCORES = 16
N_WORKERS = N_CORES * N_SUBCORES


@functools.lru_cache(maxsize=None)
def _make_sc_kernel(n_rows: int, chunk: int):
    per_w = n_rows // N_WORKERS
    n_chunks = per_w // chunk
    assert per_w % chunk == 0 and n_rows % N_WORKERS == 0
    mesh = plsc.VectorSubcoreMesh(core_axis_name="c", subcore_axis_name="s")

    @functools.partial(
        pl.kernel,
        mesh=mesh,
        out_type=jax.ShapeDtypeStruct((n_rows, D), jnp.float32),
        compiler_params=pltpu.CompilerParams(use_tc_tiling_on_sc=False),
        scratch_types=[
            pltpu.VMEM((SEQ, D), jnp.float32),
            pltpu.VMEM((D,), jnp.float32),
            pltpu.VMEM((D,), jnp.float32),
            pltpu.VMEM((chunk,), jnp.int32),
            pltpu.VMEM((chunk,), jnp.float32),
            pltpu.VMEM((chunk, D), jnp.float32),
            pltpu.SemaphoreType.DMA,
        ],
    )
    def sc_kernel(x1_hbm, x2_hbm, emb_hbm, pos_hbm, w_hbm, b_hbm, out_hbm,
                  comb_v, w_v, b_v, idx_v, x2_v, rows_v, sem):
        wid = lax.axis_index("s") * N_CORES + lax.axis_index("c")
        pltpu.sync_copy(pos_hbm.at[pl.ds(0, SEQ)], comb_v)
        pltpu.sync_copy(w_hbm, w_v)
        pltpu.sync_copy(b_hbm, b_v)

        def add_bias(r, carry):
            for k in range(NSLICE):
                sl = pl.ds(k * 16, 16)
                comb_v[r, sl] = comb_v[r, sl] + b_v[sl]
            return carry

        lax.fori_loop(0, SEQ, add_bias, 0)

        w_regs = [w_v[pl.ds(k * 16, 16)] for k in range(NSLICE)]

        def do_chunk(g, carry):
            base = wid * per_w + g * chunk
            pltpu.sync_copy(x1_hbm.at[pl.ds(base, chunk)], idx_v)
            pltpu.sync_copy(x2_hbm.at[pl.ds(base, chunk)], x2_v)
            pltpu.async_copy(emb_hbm.at[idx_v], rows_v, sem).wait()

            def row_group(i, c2):
                i16 = i * 16
                xv = x2_v[pl.ds(i16, 16)]
                for j in range(16):
                    r = i16 + j
                    pos = lax.rem(base + r, SEQ)
                    s = lax.squeeze(lax.slice(xv, (j,), (j + 1,)), (0,))
                    for k in range(NSLICE):
                        sl = pl.ds(k * 16, 16)
                        rows_v[r, sl] = (
                            rows_v[r, sl] + s * w_regs[k] + comb_v[pos, sl]
                        )
                return c2

            lax.fori_loop(0, chunk // 16, row_group, 0)
            pltpu.sync_copy(rows_v, out_hbm.at[pl.ds(base, chunk)])
            return carry

        lax.fori_loop(0, n_chunks, do_chunk, 0)

    return sc_kernel


def kernel(x1, x2, emb_table, pos_enc, mas_w, mas_b):
    b, l = x1.shape
    n_rows = b * l
    sck = _make_sc_kernel(n_rows, 512)
    out = sck(
        x1.reshape(n_rows),
        x2.reshape(n_rows),
        emb_table,
        pos_enc,
        mas_w.reshape(-1),
        mas_b,
    )
    return out.reshape(b, l, emb_table.shape[1])


# speedup vs baseline: 1.1314x; 1.1314x over previous
import functools

import jax
import jax.numpy as jnp
from jax import lax
from jax.experimental import pallas as pl
from jax.experimental.pallas import tpu as pltpu
from jax.experimental.pallas import tpu_sc as plsc

D = 64
SEQ = 200
NSLICE = D // 16
N_CORES = 2
N_SUBCORES = 16
N_WORKERS = N_CORES * N_SUBCORES


NBUF = 4


@functools.lru_cache(maxsize=None)
def _make_sc_kernel(n_rows: int, chunk: int):
    per_w = n_rows // N_WORKERS
    n_chunks = per_w // chunk
    assert per_w % chunk == 0 and n_rows % N_WORKERS == 0
    assert n_chunks % NBUF == 0 and n_chunks >= 2 * NBUF
    mesh = plsc.VectorSubcoreMesh(core_axis_name="c", subcore_axis_name="s")

    @functools.partial(
        pl.kernel,
        mesh=mesh,
        out_type=jax.ShapeDtypeStruct((n_rows, D), jnp.float32),
        compiler_params=pltpu.CompilerParams(use_tc_tiling_on_sc=False),
        scratch_types=[
            pltpu.VMEM((SEQ, D), jnp.float32),
            pltpu.VMEM((D,), jnp.float32),
            pltpu.VMEM((D,), jnp.float32),
        ]
        + [pltpu.VMEM((chunk,), jnp.int32) for _ in range(NBUF)]
        + [pltpu.VMEM((chunk,), jnp.float32) for _ in range(NBUF)]
        + [pltpu.VMEM((chunk, D), jnp.float32) for _ in range(NBUF)]
        + [pltpu.SemaphoreType.DMA for _ in range(3 * NBUF)],
    )
    def sc_kernel(x1_hbm, x2_hbm, emb_hbm, pos_hbm, w_hbm, b_hbm, out_hbm,
                  comb_v, w_v, b_v, *bufs):
        idx_v = list(bufs[0:NBUF])
        x2_v = list(bufs[NBUF:2 * NBUF])
        rows_v = list(bufs[2 * NBUF:3 * NBUF])
        lsem = list(bufs[3 * NBUF:4 * NBUF])
        gsem = list(bufs[4 * NBUF:5 * NBUF])
        osem = list(bufs[5 * NBUF:6 * NBUF])

        wid = lax.axis_index("s") * N_CORES + lax.axis_index("c")
        w0 = wid * per_w
        pltpu.sync_copy(pos_hbm.at[pl.ds(0, SEQ)], comb_v)
        pltpu.sync_copy(w_hbm, w_v)
        pltpu.sync_copy(b_hbm, b_v)

        def add_bias(r, carry):
            for k in range(NSLICE):
                sl = pl.ds(k * 16, 16)
                comb_v[r, sl] = comb_v[r, sl] + b_v[sl]
            return carry

        lax.fori_loop(0, SEQ, add_bias, 0)

        w_regs = [w_v[pl.ds(k * 16, 16)] for k in range(NSLICE)]

        def start_loads(c, b):
            base = w0 + c * chunk
            pltpu.async_copy(x1_hbm.at[pl.ds(base, chunk)], idx_v[b], lsem[b])
            pltpu.async_copy(x2_hbm.at[pl.ds(base, chunk)], x2_v[b], lsem[b])

        def wait_loads(b):
            pltpu.make_async_copy(
                x1_hbm.at[pl.ds(0, chunk)], idx_v[b], lsem[b]).wait()
            pltpu.make_async_copy(
                x2_hbm.at[pl.ds(0, chunk)], x2_v[b], lsem[b]).wait()

        def start_gather(b):
            pltpu.async_copy(emb_hbm.at[idx_v[b]], rows_v[b], gsem[b])

        def wait_gather(b):
            pltpu.make_async_copy(
                emb_hbm.at[idx_v[b]], rows_v[b], gsem[b]).wait()

        def start_store(c, b):
            base = w0 + c * chunk
            pltpu.async_copy(rows_v[b], out_hbm.at[pl.ds(base, chunk)], osem[b])

        def wait_store(b):
            pltpu.make_async_copy(
                rows_v[b], out_hbm.at[pl.ds(0, chunk)], osem[b]).wait()

        def compute(c, b):
            base = w0 + c * chunk
            rv = rows_v[b]
            xc = x2_v[b]

            def row_group(i, c2):
                i16 = i * 16
                xv = xc[pl.ds(i16, 16)]
                for j in range(16):
                    r = i16 + j
                    pos = lax.rem(base + r, SEQ)
                    s = lax.squeeze(lax.slice(xv, (j,), (j + 1,)), (0,))
                    for k in range(NSLICE):
                        sl = pl.ds(k * 16, 16)
                        rv[r, sl] = rv[r, sl] + s * w_regs[k] + comb_v[pos, sl]
                return c2

            lax.fori_loop(0, chunk // 16, row_group, 0)

        for b in range(NBUF):
            start_loads(b, b)
        wait_loads(0)
        start_gather(0)

        def steady(gg, carry):
            for b in range(NBUF):
                c = gg * NBUF + b
                bn = (b + 1) % NBUF
                @pl.when(c + 1 < n_chunks)
                def _():
                    wait_loads(bn)

                    @pl.when(c + 1 >= NBUF)
                    def _():
                        wait_store(bn)

                    start_gather(bn)

                wait_gather(b)
                compute(c, b)
                start_store(c, b)

                @pl.when(c + NBUF < n_chunks)
                def _():
                    start_loads(c + NBUF, b)

            return carry

        lax.fori_loop(0, n_chunks // NBUF, steady, 0)
        for b in range(NBUF):
            wait_store(b)

    return sc_kernel


def kernel(x1, x2, emb_table, pos_enc, mas_w, mas_b):
    b, l = x1.shape
    n_rows = b * l
    sck = _make_sc_kernel(n_rows, 320)
    out = sck(
        x1.reshape(n_rows),
        x2.reshape(n_rows),
        emb_table,
        pos_enc,
        mas_w.reshape(-1),
        mas_b,
    )
    return out.reshape(b, l, emb_table.shape[1])


# speedup vs baseline: 1.3979x; 1.2355x over previous
import functools

import jax
import jax.numpy as jnp
from jax import lax
from jax.experimental import pallas as pl
from jax.experimental.pallas import tpu as pltpu
from jax.experimental.pallas import tpu_sc as plsc

D = 64
SEQ = 200
NSLICE = D // 16
N_CORES = 2
N_SUBCORES = 16
N_WORKERS = N_CORES * N_SUBCORES

NBUF = 4


@functools.lru_cache(maxsize=None)
def _make_sc_kernel(n_batch: int):
    per_w = n_batch // N_WORKERS
    assert n_batch % N_WORKERS == 0
    assert per_w % NBUF == 0 and per_w >= 2 * NBUF
    mesh = plsc.VectorSubcoreMesh(core_axis_name="c", subcore_axis_name="s")

    @functools.partial(
        pl.kernel,
        mesh=mesh,
        out_type=jax.ShapeDtypeStruct((n_batch, SEQ, D), jnp.float32),
        compiler_params=pltpu.CompilerParams(use_tc_tiling_on_sc=False),
        scratch_types=[
            pltpu.VMEM((SEQ, D), jnp.float32),
            pltpu.VMEM((D,), jnp.float32),
            pltpu.VMEM((D,), jnp.float32),
        ]
        + [pltpu.VMEM((SEQ,), jnp.int32) for _ in range(NBUF)]
        + [pltpu.VMEM((SEQ,), jnp.float32) for _ in range(NBUF)]
        + [pltpu.VMEM((SEQ, D), jnp.float32) for _ in range(NBUF)]
        + [pltpu.SemaphoreType.DMA for _ in range(3 * NBUF)],
    )
    def sc_kernel(x1_hbm, x2_hbm, emb_hbm, pos_hbm, w_hbm, b_hbm, out_hbm,
                  comb_v, w_v, b_v, *bufs):
        idx_v = list(bufs[0:NBUF])
        x2_v = list(bufs[NBUF:2 * NBUF])
        rows_v = list(bufs[2 * NBUF:3 * NBUF])
        lsem = list(bufs[3 * NBUF:4 * NBUF])
        gsem = list(bufs[4 * NBUF:5 * NBUF])
        osem = list(bufs[5 * NBUF:6 * NBUF])

        wid = lax.axis_index("s") * N_CORES + lax.axis_index("c")
        b0 = wid * per_w
        pltpu.sync_copy(pos_hbm.at[pl.ds(0, SEQ)], comb_v)
        pltpu.sync_copy(w_hbm, w_v)
        pltpu.sync_copy(b_hbm, b_v)

        def add_bias(r, carry):
            for k in range(NSLICE):
                sl = pl.ds(k * 16, 16)
                comb_v[r, sl] = comb_v[r, sl] + b_v[sl]
            return carry

        lax.fori_loop(0, SEQ, add_bias, 0)

        w_regs = [w_v[pl.ds(k * 16, 16)] for k in range(NSLICE)]

        def start_loads(c, b):
            row = b0 + c
            pltpu.async_copy(x1_hbm.at[row], idx_v[b], lsem[b])
            pltpu.async_copy(x2_hbm.at[row], x2_v[b], lsem[b])

        def wait_loads(b):
            pltpu.make_async_copy(x1_hbm.at[0], idx_v[b], lsem[b]).wait()
            pltpu.make_async_copy(x2_hbm.at[0], x2_v[b], lsem[b]).wait()

        def start_gather(b):
            pltpu.async_copy(emb_hbm.at[idx_v[b]], rows_v[b], gsem[b])

        def wait_gather(b):
            pltpu.make_async_copy(
                emb_hbm.at[idx_v[b]], rows_v[b], gsem[b]).wait()

        def start_store(c, b):
            row = b0 + c
            pltpu.async_copy(rows_v[b], out_hbm.at[row], osem[b])

        def wait_store(b):
            pltpu.make_async_copy(rows_v[b], out_hbm.at[0], osem[b]).wait()

        def do_rows(rv, xv_ref, r_base, jrange):
            xv = xv_ref
            for j in jrange:
                r = r_base + j
                s = lax.squeeze(lax.slice(xv, (j,), (j + 1,)), (0,))
                for k in range(NSLICE):
                    sl = pl.ds(k * 16, 16)
                    rv[r, sl] = rv[r, sl] + s * w_regs[k] + comb_v[r, sl]

        def compute(b):
            rv = rows_v[b]
            xc = x2_v[b]

            def group(i, c2):
                do_rows(rv, xc[pl.ds(i * 16, 16)], i * 16, range(16))
                return c2

            lax.fori_loop(0, (SEQ // 16), group, 0)
            do_rows(rv, xc[pl.ds(SEQ - 16, 16)], SEQ - 16, range(8, 16))

        for b in range(NBUF):
            start_loads(b, b)
        wait_loads(0)
        start_gather(0)

        def steady(gg, carry):
            for b in range(NBUF):
                c = gg * NBUF + b
                bn = (b + 1) % NBUF
                @pl.when(c + 1 < per_w)
                def _():
                    wait_loads(bn)

                    @pl.when(c + 1 >= NBUF)
                    def _():
                        wait_store(bn)

                    start_gather(bn)

                wait_gather(b)
                compute(b)
                start_store(c, b)

                @pl.when(c + NBUF < per_w)
                def _():
                    start_loads(c + NBUF, b)

            return carry

        lax.fori_loop(0, per_w // NBUF, steady, 0)
        for b in range(NBUF):
            wait_store(b)

    return sc_kernel


def kernel(x1, x2, emb_table, pos_enc, mas_w, mas_b):
    b, l = x1.shape
    sck = _make_sc_kernel(b)
    return sck(x1, x2, emb_table, pos_enc, mas_w.reshape(-1), mas_b)


# speedup vs baseline: 1.5424x; 1.1034x over previous
import functools

import jax
import jax.numpy as jnp
from jax import lax
from jax.experimental import pallas as pl
from jax.experimental.pallas import tpu as pltpu
from jax.experimental.pallas import tpu_sc as plsc

VOCAB = 1000000
D = 64
SEQ = 200
N_CORES = 2
N_SUBCORES = 16
N_WORKERS = N_CORES * N_SUBCORES

NBUF = 4

TBLK = 2048
BBLK = 512
LBLK = 8


def _transpose_table(emb_t):
    def body(in_ref, out_ref):
        t = in_ref[...].T
        out_ref[...] = jnp.concatenate([t, t], axis=1)

    grid = (pl.cdiv(VOCAB, TBLK),)
    return pl.pallas_call(
        body,
        grid=grid,
        in_specs=[pl.BlockSpec((D, TBLK), lambda i: (0, i))],
        out_specs=pl.BlockSpec((TBLK, 2 * D), lambda i: (i, 0)),
        out_shape=jax.ShapeDtypeStruct((VOCAB, 2 * D), jnp.float32),
    )(emb_t)


@functools.lru_cache(maxsize=None)
def _make_gather(n_batch: int):
    per_w = n_batch // N_WORKERS
    assert n_batch % N_WORKERS == 0
    assert per_w % NBUF == 0 and per_w >= 2 * NBUF
    mesh = plsc.VectorSubcoreMesh(core_axis_name="c", subcore_axis_name="s")

    @functools.partial(
        pl.kernel,
        mesh=mesh,
        out_type=jax.ShapeDtypeStruct((n_batch, SEQ, 2 * D), jnp.float32),
        compiler_params=pltpu.CompilerParams(use_tc_tiling_on_sc=True),
        scratch_types=[pltpu.VMEM((SEQ,), jnp.int32) for _ in range(NBUF)]
        + [pltpu.VMEM((SEQ, 2 * D), jnp.float32) for _ in range(NBUF)]
        + [pltpu.SemaphoreType.DMA for _ in range(3 * NBUF)],
    )
    def gather_kernel(x1_hbm, emb_hbm, out_hbm, *bufs):
        idx_v = list(bufs[0:NBUF])
        rows_v = list(bufs[NBUF:2 * NBUF])
        lsem = list(bufs[2 * NBUF:3 * NBUF])
        gsem = list(bufs[3 * NBUF:4 * NBUF])
        osem = list(bufs[4 * NBUF:5 * NBUF])

        wid = lax.axis_index("s") * N_CORES + lax.axis_index("c")
        b0 = wid * per_w

        def start_load(c, b):
            pltpu.async_copy(x1_hbm.at[b0 + c], idx_v[b], lsem[b])

        def wait_load(b):
            pltpu.make_async_copy(x1_hbm.at[0], idx_v[b], lsem[b]).wait()

        def start_gather(b):
            pltpu.async_copy(emb_hbm.at[idx_v[b]], rows_v[b], gsem[b])

        def wait_gather(b):
            pltpu.make_async_copy(
                emb_hbm.at[idx_v[b]], rows_v[b], gsem[b]).wait()

        def start_store(c, b):
            pltpu.async_copy(rows_v[b], out_hbm.at[b0 + c], osem[b])

        def wait_store(b):
            pltpu.make_async_copy(rows_v[b], out_hbm.at[0], osem[b]).wait()

        for b in range(NBUF):
            start_load(b, b)
        wait_load(0)
        start_gather(0)

        def steady(gg, carry):
            for b in range(NBUF):
                c = gg * NBUF + b
                bn = (b + 1) % NBUF
                @pl.when(c + 1 < per_w)
                def _():
                    wait_load(bn)

                    @pl.when(c + 1 >= NBUF)
                    def _():
                        wait_store(bn)

                    start_gather(bn)

                wait_gather(b)
                start_store(c, b)

                @pl.when(c + NBUF < per_w)
                def _():
                    start_load(c + NBUF, b)

            return carry

        lax.fori_loop(0, per_w // NBUF, steady, 0)
        for b in range(NBUF):
            wait_store(b)

    return gather_kernel


def _finalize(gath, x2t, pos, w2, b2, n_batch):
    def body(g_ref, x2_ref, pos_ref, w_ref, b_ref, out_ref):
        x2b = x2_ref[...]
        w = w_ref[0]
        bias = b_ref[0]
        pp = pos_ref[...] + bias[None, :]
        for l in range(LBLK):
            t_l = g_ref[:, l, :D].T
            out_ref[l] = t_l + x2b[l][None, :] * w[:, None] + pp[l][:, None]

    grid = (SEQ // LBLK, n_batch // BBLK)
    return pl.pallas_call(
        body,
        grid=grid,
        in_specs=[
            pl.BlockSpec((BBLK, LBLK, 2 * D), lambda i, j: (j, i, 0)),

            pl.BlockSpec((LBLK, BBLK), lambda i, j: (i, j)),
            pl.BlockSpec((LBLK, D), lambda i, j: (i, 0)),
            pl.BlockSpec((1, D), lambda i, j: (0, 0)),
            pl.BlockSpec((1, D), lambda i, j: (0, 0)),
        ],
        out_specs=pl.BlockSpec((LBLK, D, BBLK), lambda i, j: (i, 0, j)),
        out_shape=jax.ShapeDtypeStruct((SEQ, D, n_batch), jnp.float32),
    )(gath, x2t, pos, w2, b2)


def kernel(x1, x2, emb_table, pos_enc, mas_w, mas_b):
    n_batch, l = x1.shape
    emb_rm = _transpose_table(jnp.transpose(emb_table))
    gath = _make_gather(n_batch)(x1, emb_rm)
    out_t = _finalize(
        gath,
        jnp.transpose(x2),
        pos_enc[:SEQ],
        mas_w.reshape(1, D),
        mas_b.reshape(1, D),
        n_batch,
    )
    return jnp.transpose(out_t, (2, 0, 1))


# speedup vs baseline: 1.6086x; 1.0429x over previous
import functools

import jax
import jax.numpy as jnp
from jax import lax
from jax.experimental import pallas as pl
from jax.experimental.pallas import tpu as pltpu
from jax.experimental.pallas import tpu_sc as plsc

VOCAB = 1000000
D = 64
SEQ = 200
N_CORES = 2
N_SUBCORES = 16
N_WORKERS = N_CORES * N_SUBCORES

NBUF = 4

TBLK = 2048
BBLK = 512
LBLK = 8


def _transpose_table(emb_t):
    def body(in_ref, out_ref):
        x = in_ref[...]
        t = jax.lax.dot_general(
            x, jnp.eye(D, dtype=jnp.float32),
            dimension_numbers=(((0,), (0,)), ((), ())),
            preferred_element_type=jnp.float32,
        )
        out_ref[:, :D] = t
        out_ref[:, D:] = t

    grid = (pl.cdiv(VOCAB, TBLK),)
    return pl.pallas_call(
        body,
        grid=grid,
        in_specs=[pl.BlockSpec((D, TBLK), lambda i: (0, i))],
        out_specs=pl.BlockSpec((TBLK, 2 * D), lambda i: (i, 0)),
        out_shape=jax.ShapeDtypeStruct((VOCAB, 2 * D), jnp.float32),
    )(emb_t)


@functools.lru_cache(maxsize=None)
def _make_gather(n_batch: int):
    per_w = n_batch // N_WORKERS
    assert n_batch % N_WORKERS == 0
    assert per_w % NBUF == 0 and per_w >= 2 * NBUF
    mesh = plsc.VectorSubcoreMesh(core_axis_name="c", subcore_axis_name="s")

    @functools.partial(
        pl.kernel,
        mesh=mesh,
        out_type=jax.ShapeDtypeStruct((n_batch, SEQ, 2 * D), jnp.float32),
        compiler_params=pltpu.CompilerParams(use_tc_tiling_on_sc=True),
        scratch_types=[pltpu.VMEM((SEQ,), jnp.int32) for _ in range(NBUF)]
        + [pltpu.VMEM((SEQ, 2 * D), jnp.float32) for _ in range(NBUF)]
        + [pltpu.SemaphoreType.DMA for _ in range(3 * NBUF)],
    )
    def gather_kernel(x1_hbm, emb_hbm, out_hbm, *bufs):
        idx_v = list(bufs[0:NBUF])
        rows_v = list(bufs[NBUF:2 * NBUF])
        lsem = list(bufs[2 * NBUF:3 * NBUF])
        gsem = list(bufs[3 * NBUF:4 * NBUF])
        osem = list(bufs[4 * NBUF:5 * NBUF])

        wid = lax.axis_index("s") * N_CORES + lax.axis_index("c")
        b0 = wid * per_w

        def start_load(c, b):
            pltpu.async_copy(x1_hbm.at[b0 + c], idx_v[b], lsem[b])

        def wait_load(b):
            pltpu.make_async_copy(x1_hbm.at[0], idx_v[b], lsem[b]).wait()

        def start_gather(b):
            pltpu.async_copy(emb_hbm.at[idx_v[b]], rows_v[b], gsem[b])

        def wait_gather(b):
            pltpu.make_async_copy(
                emb_hbm.at[idx_v[b]], rows_v[b], gsem[b]).wait()

        def start_store(c, b):
            pltpu.async_copy(rows_v[b], out_hbm.at[b0 + c], osem[b])

        def wait_store(b):
            pltpu.make_async_copy(rows_v[b], out_hbm.at[0], osem[b]).wait()

        for b in range(NBUF):
            start_load(b, b)
        wait_load(0)
        start_gather(0)

        def steady(gg, carry):
            for b in range(NBUF):
                c = gg * NBUF + b
                bn = (b + 1) % NBUF
                @pl.when(c + 1 < per_w)
                def _():
                    wait_load(bn)

                    @pl.when(c + 1 >= NBUF)
                    def _():
                        wait_store(bn)

                    start_gather(bn)

                wait_gather(b)
                start_store(c, b)

                @pl.when(c + NBUF < per_w)
                def _():
                    start_load(c + NBUF, b)

            return carry

        lax.fori_loop(0, per_w // NBUF, steady, 0)
        for b in range(NBUF):
            wait_store(b)

    return gather_kernel


def _finalize(gath, x2t, pos, w2, b2, n_batch):
    def body(g_ref, x2_ref, pos_ref, w_ref, b_ref, out_ref):
        x2b = x2_ref[...]
        w = w_ref[0]
        bias = b_ref[0]
        pp = pos_ref[...] + bias[None, :]
        g2 = g_ref[...].reshape(BBLK, LBLK * 2 * D)
        t = jnp.transpose(g2)
        for l in range(LBLK):
            t_l = t[l * 2 * D:l * 2 * D + D]
            out_ref[l] = t_l + x2b[l][None, :] * w[:, None] + pp[l][:, None]

    grid = (SEQ // LBLK, n_batch // BBLK)
    return pl.pallas_call(
        body,
        grid=grid,
        in_specs=[
            pl.BlockSpec((BBLK, LBLK, 2 * D), lambda i, j: (j, i, 0)),

            pl.BlockSpec((LBLK, BBLK), lambda i, j: (i, j)),
            pl.BlockSpec((LBLK, D), lambda i, j: (i, 0)),
            pl.BlockSpec((1, D), lambda i, j: (0, 0)),
            pl.BlockSpec((1, D), lambda i, j: (0, 0)),
        ],
        out_specs=pl.BlockSpec((LBLK, D, BBLK), lambda i, j: (i, 0, j)),
        out_shape=jax.ShapeDtypeStruct((SEQ, D, n_batch), jnp.float32),
    )(gath, x2t, pos, w2, b2)


def kernel(x1, x2, emb_table, pos_enc, mas_w, mas_b):
    n_batch, l = x1.shape
    emb_rm = _transpose_table(jnp.transpose(emb_table))
    gath = _make_gather(n_batch)(x1, emb_rm)
    out_t = _finalize(
        gath,
        jnp.transpose(x2),
        pos_enc[:SEQ],
        mas_w.reshape(1, D),
        mas_b.reshape(1, D),
        n_batch,
    )
    return jnp.transpose(out_t, (2, 0, 1))


# speedup vs baseline: 1.9516x; 1.2133x over previous
import functools

import jax
import jax.numpy as jnp
from jax import lax
from jax.experimental import pallas as pl
from jax.experimental.pallas import tpu as pltpu
from jax.experimental.pallas import tpu_sc as plsc

VOCAB = 1000000
D = 64
SEQ = 200
N_CORES = 2
N_SUBCORES = 16
N_WORKERS = N_CORES * N_SUBCORES

NBUF = 4

TBLK = 8192
BBLK = 512
LBLK = 8


def _transpose_table(emb_t):
    def body(in_ref, out_ref):
        x = in_ref[...]
        t = jax.lax.dot_general(
            x, jnp.eye(D, dtype=jnp.float32),
            dimension_numbers=(((0,), (0,)), ((), ())),
            preferred_element_type=jnp.float32,
        )
        out_ref[:, :D] = t
        out_ref[:, D:] = t

    grid = (pl.cdiv(VOCAB, TBLK),)
    return pl.pallas_call(
        body,
        grid=grid,
        in_specs=[pl.BlockSpec((D, TBLK), lambda i: (0, i))],
        out_specs=pl.BlockSpec((TBLK, 2 * D), lambda i: (i, 0)),
        out_shape=jax.ShapeDtypeStruct((VOCAB, 2 * D), jnp.float32),
    )(emb_t)


@functools.lru_cache(maxsize=None)
def _make_gather(n_batch: int):
    per_w = n_batch // N_WORKERS
    assert n_batch % N_WORKERS == 0
    assert per_w % NBUF == 0 and per_w >= 2 * NBUF
    mesh = plsc.VectorSubcoreMesh(core_axis_name="c", subcore_axis_name="s")

    @functools.partial(
        pl.kernel,
        mesh=mesh,
        out_type=jax.ShapeDtypeStruct((n_batch, SEQ, 2 * D), jnp.float32),
        compiler_params=pltpu.CompilerParams(use_tc_tiling_on_sc=True),
        scratch_types=[pltpu.VMEM((SEQ,), jnp.int32) for _ in range(NBUF)]
        + [pltpu.VMEM((SEQ, 2 * D), jnp.float32) for _ in range(NBUF)]
        + [pltpu.SemaphoreType.DMA for _ in range(3 * NBUF)],
    )
    def gather_kernel(x1_hbm, emb_hbm, out_hbm, *bufs):
        idx_v = list(bufs[0:NBUF])
        rows_v = list(bufs[NBUF:2 * NBUF])
        lsem = list(bufs[2 * NBUF:3 * NBUF])
        gsem = list(bufs[3 * NBUF:4 * NBUF])
        osem = list(bufs[4 * NBUF:5 * NBUF])

        wid = lax.axis_index("s") * N_CORES + lax.axis_index("c")
        b0 = wid * per_w

        def start_load(c, b):
            pltpu.async_copy(x1_hbm.at[b0 + c], idx_v[b], lsem[b])

        def wait_load(b):
            pltpu.make_async_copy(x1_hbm.at[0], idx_v[b], lsem[b]).wait()

        def start_gather(b):
            pltpu.async_copy(emb_hbm.at[idx_v[b]], rows_v[b], gsem[b])

        def wait_gather(b):
            pltpu.make_async_copy(
                emb_hbm.at[idx_v[b]], rows_v[b], gsem[b]).wait()

        def start_store(c, b):
            pltpu.async_copy(rows_v[b], out_hbm.at[b0 + c], osem[b])

        def wait_store(b):
            pltpu.make_async_copy(rows_v[b], out_hbm.at[0], osem[b]).wait()

        for b in range(NBUF):
            start_load(b, b)
        wait_load(0)
        start_gather(0)

        def steady(gg, carry):
            for b in range(NBUF):
                c = gg * NBUF + b
                bn = (b + 1) % NBUF
                @pl.when(c + 1 < per_w)
                def _():
                    wait_load(bn)

                    @pl.when(c + 1 >= NBUF)
                    def _():
                        wait_store(bn)

                    start_gather(bn)

                wait_gather(b)
                start_store(c, b)

                @pl.when(c + NBUF < per_w)
                def _():
                    start_load(c + NBUF, b)

            return carry

        lax.fori_loop(0, per_w // NBUF, steady, 0)
        for b in range(NBUF):
            wait_store(b)

    return gather_kernel


def _finalize(gath, x2t, pos, w2, b2, n_batch):
    def body(g_ref, x2_ref, pos_ref, w_ref, b_ref, out_ref):
        x2b = x2_ref[...]
        w = w_ref[0]
        bias = b_ref[0]
        pp = pos_ref[...] + bias[None, :]
        g2 = g_ref[...].reshape(BBLK, LBLK * 2 * D)
        t = jnp.transpose(g2)
        for l in range(LBLK):
            t_l = t[l * 2 * D:l * 2 * D + D]
            out_ref[l] = t_l + x2b[l][None, :] * w[:, None] + pp[l][:, None]

    grid = (SEQ // LBLK, n_batch // BBLK)
    return pl.pallas_call(
        body,
        grid=grid,
        in_specs=[
            pl.BlockSpec((BBLK, LBLK, 2 * D), lambda i, j: (j, i, 0)),

            pl.BlockSpec((LBLK, BBLK), lambda i, j: (i, j)),
            pl.BlockSpec((LBLK, D), lambda i, j: (i, 0)),
            pl.BlockSpec((1, D), lambda i, j: (0, 0)),
            pl.BlockSpec((1, D), lambda i, j: (0, 0)),
        ],
        out_specs=pl.BlockSpec((LBLK, D, BBLK), lambda i, j: (i, 0, j)),
        out_shape=jax.ShapeDtypeStruct((SEQ, D, n_batch), jnp.float32),
    )(gath, x2t, pos, w2, b2)


def kernel(x1, x2, emb_table, pos_enc, mas_w, mas_b):
    n_batch, l = x1.shape
    emb_rm = _transpose_table(jnp.transpose(emb_table))
    gath = _make_gather(n_batch)(x1, emb_rm)
    out_t = _finalize(
        gath,
        jnp.transpose(x2),
        pos_enc[:SEQ],
        mas_w.reshape(1, D),
        mas_b.reshape(1, D),
        n_batch,
    )
    return jnp.transpose(out_t, (2, 0, 1))


# speedup vs baseline: 1.9951x; 1.0223x over previous
import functools

import jax
import jax.numpy as jnp
from jax import lax
from jax.experimental import pallas as pl
from jax.experimental.pallas import tpu as pltpu
from jax.experimental.pallas import tpu_sc as plsc

VOCAB = 1000000
D = 64
SEQ = 200
N_CORES = 2
N_SUBCORES = 16
N_WORKERS = N_CORES * N_SUBCORES

NBUF = 4

TBLK = 8192
BBLK = 512
LBLK = 8


def _transpose_table(emb_t):
    def body(in_ref, out_ref):
        x = in_ref[...]
        t = jax.lax.dot_general(
            x, jnp.eye(D, dtype=jnp.float32),
            dimension_numbers=(((0,), (0,)), ((), ())),
            preferred_element_type=jnp.float32,
        )
        out_ref[:, :D] = t
        out_ref[:, D:] = t

    grid = (pl.cdiv(VOCAB, TBLK),)
    return pl.pallas_call(
        body,
        grid=grid,
        in_specs=[pl.BlockSpec((D, TBLK), lambda i: (0, i))],
        out_specs=pl.BlockSpec((TBLK, 2 * D), lambda i: (i, 0)),
        out_shape=jax.ShapeDtypeStruct((VOCAB, 2 * D), jnp.float32),
    )(emb_t)


@functools.lru_cache(maxsize=None)
def _make_gather(n_batch: int, l_len: int):
    per_w = n_batch // N_WORKERS
    assert n_batch % N_WORKERS == 0
    assert per_w % NBUF == 0 and per_w >= 2 * NBUF
    mesh = plsc.VectorSubcoreMesh(core_axis_name="c", subcore_axis_name="s")

    @functools.partial(
        pl.kernel,
        mesh=mesh,
        out_type=jax.ShapeDtypeStruct((n_batch, l_len, 2 * D), jnp.float32),
        compiler_params=pltpu.CompilerParams(use_tc_tiling_on_sc=True),
        scratch_types=[pltpu.VMEM((l_len,), jnp.int32) for _ in range(NBUF)]
        + [pltpu.VMEM((l_len, 2 * D), jnp.float32) for _ in range(NBUF)]
        + [pltpu.SemaphoreType.DMA for _ in range(3 * NBUF)],
    )
    def gather_kernel(x1_hbm, emb_hbm, out_hbm, *bufs):
        idx_v = list(bufs[0:NBUF])
        rows_v = list(bufs[NBUF:2 * NBUF])
        lsem = list(bufs[2 * NBUF:3 * NBUF])
        gsem = list(bufs[3 * NBUF:4 * NBUF])
        osem = list(bufs[4 * NBUF:5 * NBUF])

        wid = lax.axis_index("s") * N_CORES + lax.axis_index("c")
        b0 = wid * per_w

        def start_load(c, b):
            pltpu.async_copy(x1_hbm.at[b0 + c], idx_v[b], lsem[b])

        def wait_load(b):
            pltpu.make_async_copy(x1_hbm.at[0], idx_v[b], lsem[b]).wait()

        def start_gather(b):
            pltpu.async_copy(emb_hbm.at[idx_v[b]], rows_v[b], gsem[b])

        def wait_gather(b):
            pltpu.make_async_copy(
                emb_hbm.at[idx_v[b]], rows_v[b], gsem[b]).wait()

        def start_store(c, b):
            pltpu.async_copy(rows_v[b], out_hbm.at[b0 + c], osem[b])

        def wait_store(b):
            pltpu.make_async_copy(rows_v[b], out_hbm.at[0], osem[b]).wait()

        for b in range(NBUF):
            start_load(b, b)
        wait_load(0)
        start_gather(0)

        def steady(gg, carry):
            for b in range(NBUF):
                c = gg * NBUF + b
                bn = (b + 1) % NBUF
                @pl.when(c + 1 < per_w)
                def _():
                    wait_load(bn)

                    @pl.when(c + 1 >= NBUF)
                    def _():
                        wait_store(bn)

                    start_gather(bn)

                wait_gather(b)
                start_store(c, b)

                @pl.when(c + NBUF < per_w)
                def _():
                    start_load(c + NBUF, b)

            return carry

        lax.fori_loop(0, per_w // NBUF, steady, 0)
        for b in range(NBUF):
            wait_store(b)

    return gather_kernel


def _finalize_part(gath, x2t_p, pos_p, w2, b2, n_batch, l_off, l_len, prev):
    def body(*refs):
        g_ref, x2_ref, pos_ref, w_ref, b_ref, out_ref = refs[-6:]
        x2b = x2_ref[...]
        w = w_ref[0]
        bias = b_ref[0]
        pp = pos_ref[...] + bias[None, :]
        g2 = g_ref[...].reshape(BBLK, LBLK * 2 * D)
        t = jnp.transpose(g2)
        for l in range(LBLK):
            t_l = t[l * 2 * D:l * 2 * D + D]
            out_ref[l] = t_l + x2b[l][None, :] * w[:, None] + pp[l][:, None]

    i0 = l_off // LBLK
    grid = (l_len // LBLK, n_batch // BBLK)
    in_specs = [
        pl.BlockSpec((BBLK, LBLK, 2 * D), lambda i, j: (j, i, 0)),
        pl.BlockSpec((LBLK, BBLK), lambda i, j: (i, j)),
        pl.BlockSpec((LBLK, D), lambda i, j: (i, 0)),
        pl.BlockSpec((1, D), lambda i, j: (0, 0)),
        pl.BlockSpec((1, D), lambda i, j: (0, 0)),
    ]
    args = (gath, x2t_p, pos_p, w2, b2)
    aliases = {}
    if prev is not None:
        in_specs = [pl.BlockSpec(memory_space=pl.ANY)] + in_specs
        args = (prev,) + args
        aliases = {0: 0}
    return pl.pallas_call(
        body,
        grid=grid,
        in_specs=in_specs,
        out_specs=pl.BlockSpec((LBLK, D, BBLK), lambda i, j: (i + i0, 0, j)),
        out_shape=jax.ShapeDtypeStruct((SEQ, D, n_batch), jnp.float32),
        input_output_aliases=aliases,
    )(*args)


def kernel(x1, x2, emb_table, pos_enc, mas_w, mas_b):
    n_batch, l = x1.shape
    emb_rm = _transpose_table(jnp.transpose(emb_table))
    x2t = jnp.transpose(x2)
    w2 = mas_w.reshape(1, D)
    b2 = mas_b.reshape(1, D)
    parts = ((0, 48), (48, 48), (96, 48), (144, 56))
    gaths = [
        _make_gather(n_batch, ln)(
            lax.slice(x1, (0, off), (n_batch, off + ln)), emb_rm)
        for off, ln in parts
    ]
    out_t = None
    for (off, ln), g in zip(parts, gaths):
        out_t = _finalize_part(
            g,
            lax.slice(x2t, (off, 0), (off + ln, n_batch)),
            lax.slice(pos_enc, (off, 0), (off + ln, D)),
            w2, b2, n_batch, off, ln, out_t,
        )
    return jnp.transpose(out_t, (2, 0, 1))


# speedup vs baseline: 2.1107x; 1.0579x over previous
import functools

import jax
import jax.numpy as jnp
from jax import lax
from jax.experimental import pallas as pl
from jax.experimental.pallas import tpu as pltpu
from jax.experimental.pallas import tpu_sc as plsc

VOCAB = 1000000
D = 64
SEQ = 200
N_CORES = 2
N_SUBCORES = 16
N_WORKERS = N_CORES * N_SUBCORES

NBUF = 4

TBLK = 8192
BBLK = 512
LBLK = 8


def _transpose_table(emb_t):
    def body(in_ref, out_ref):
        x = in_ref[...]
        t = jax.lax.dot_general(
            x, jnp.eye(D, dtype=jnp.float32),
            dimension_numbers=(((0,), (0,)), ((), ())),
            preferred_element_type=jnp.float32,
        )
        out_ref[:, :D] = t

    grid = (pl.cdiv(VOCAB, TBLK),)
    return pl.pallas_call(
        body,
        grid=grid,
        in_specs=[pl.BlockSpec((D, TBLK), lambda i: (0, i))],
        out_specs=pl.BlockSpec((TBLK, 2 * D), lambda i: (i, 0)),
        out_shape=jax.ShapeDtypeStruct((VOCAB, 2 * D), jnp.float32),
    )(emb_t)


@functools.lru_cache(maxsize=None)
def _make_gather(n_batch: int, l_len: int):
    per_w = n_batch // N_WORKERS
    assert n_batch % N_WORKERS == 0
    assert per_w % NBUF == 0 and per_w >= 2 * NBUF
    mesh = plsc.VectorSubcoreMesh(core_axis_name="c", subcore_axis_name="s")

    @functools.partial(
        pl.kernel,
        mesh=mesh,
        out_type=jax.ShapeDtypeStruct((n_batch, l_len, 2 * D), jnp.float32),
        compiler_params=pltpu.CompilerParams(use_tc_tiling_on_sc=True),
        scratch_types=[pltpu.VMEM((l_len,), jnp.int32) for _ in range(NBUF)]
        + [pltpu.VMEM((l_len, 2 * D), jnp.float32) for _ in range(NBUF)]
        + [pltpu.SemaphoreType.DMA for _ in range(3 * NBUF)],
    )
    def gather_kernel(x1_hbm, emb_hbm, out_hbm, *bufs):
        idx_v = list(bufs[0:NBUF])
        rows_v = list(bufs[NBUF:2 * NBUF])
        lsem = list(bufs[2 * NBUF:3 * NBUF])
        gsem = list(bufs[3 * NBUF:4 * NBUF])
        osem = list(bufs[4 * NBUF:5 * NBUF])

        wid = lax.axis_index("s") * N_CORES + lax.axis_index("c")
        b0 = wid * per_w

        def start_load(c, b):
            pltpu.async_copy(x1_hbm.at[b0 + c], idx_v[b], lsem[b])

        def wait_load(b):
            pltpu.make_async_copy(x1_hbm.at[0], idx_v[b], lsem[b]).wait()

        def start_gather(b):
            pltpu.async_copy(emb_hbm.at[idx_v[b]], rows_v[b], gsem[b])

        def wait_gather(b):
            pltpu.make_async_copy(
                emb_hbm.at[idx_v[b]], rows_v[b], gsem[b]).wait()

        def start_store(c, b):
            pltpu.async_copy(rows_v[b], out_hbm.at[b0 + c], osem[b])

        def wait_store(b):
            pltpu.make_async_copy(rows_v[b], out_hbm.at[0], osem[b]).wait()

        for b in range(NBUF):
            start_load(b, b)
        wait_load(0)
        start_gather(0)

        def steady(gg, carry):
            for b in range(NBUF):
                c = gg * NBUF + b
                bn = (b + 1) % NBUF
                @pl.when(c + 1 < per_w)
                def _():
                    wait_load(bn)

                    @pl.when(c + 1 >= NBUF)
                    def _():
                        wait_store(bn)

                    start_gather(bn)

                wait_gather(b)
                start_store(c, b)

                @pl.when(c + NBUF < per_w)
                def _():
                    start_load(c + NBUF, b)

            return carry

        lax.fori_loop(0, per_w // NBUF, steady, 0)
        for b in range(NBUF):
            wait_store(b)

    return gather_kernel


def _finalize_part(gath, x2t_p, pos_p, w2, b2, n_batch, l_off, l_len, prev):
    def body(*refs):
        g_ref, x2_ref, pos_ref, w_ref, b_ref, out_ref = refs[-6:]
        x2b = x2_ref[...]
        w = w_ref[0]
        bias = b_ref[0]
        pp = pos_ref[...] + bias[None, :]
        g2 = g_ref[...].reshape(BBLK, LBLK * 2 * D)
        t = jnp.transpose(g2)
        for l in range(LBLK):
            t_l = t[l * 2 * D:l * 2 * D + D]
            out_ref[l] = t_l + x2b[l][None, :] * w[:, None] + pp[l][:, None]

    i0 = l_off // LBLK
    grid = (l_len // LBLK, n_batch // BBLK)
    in_specs = [
        pl.BlockSpec((BBLK, LBLK, 2 * D), lambda i, j: (j, i, 0)),
        pl.BlockSpec((LBLK, BBLK), lambda i, j: (i, j)),
        pl.BlockSpec((LBLK, D), lambda i, j: (i, 0)),
        pl.BlockSpec((1, D), lambda i, j: (0, 0)),
        pl.BlockSpec((1, D), lambda i, j: (0, 0)),
    ]
    args = (gath, x2t_p, pos_p, w2, b2)
    aliases = {}
    if prev is not None:
        in_specs = [pl.BlockSpec(memory_space=pl.ANY)] + in_specs
        args = (prev,) + args
        aliases = {0: 0}
    return pl.pallas_call(
        body,
        grid=grid,
        in_specs=in_specs,
        out_specs=pl.BlockSpec((LBLK, D, BBLK), lambda i, j: (i + i0, 0, j)),
        out_shape=jax.ShapeDtypeStruct((SEQ, D, n_batch), jnp.float32),
        input_output_aliases=aliases,
    )(*args)


def kernel(x1, x2, emb_table, pos_enc, mas_w, mas_b):
    n_batch, l = x1.shape
    emb_rm = _transpose_table(jnp.transpose(emb_table))
    x2t = jnp.transpose(x2)
    w2 = mas_w.reshape(1, D)
    b2 = mas_b.reshape(1, D)
    parts = ((0, 48), (48, 48), (96, 48), (144, 56))
    gaths = [
        _make_gather(n_batch, ln)(
            lax.slice(x1, (0, off), (n_batch, off + ln)), emb_rm)
        for off, ln in parts
    ]
    out_t = None
    for (off, ln), g in zip(parts, gaths):
        out_t = _finalize_part(
            g,
            lax.slice(x2t, (off, 0), (off + ln, n_batch)),
            lax.slice(pos_enc, (off, 0), (off + ln, D)),
            w2, b2, n_batch, off, ln, out_t,
        )
    return jnp.transpose(out_t, (2, 0, 1))


# speedup vs baseline: 2.2013x; 1.0429x over previous
import functools

import jax
import jax.numpy as jnp
from jax import lax
from jax.experimental import pallas as pl
from jax.experimental.pallas import tpu as pltpu
from jax.experimental.pallas import tpu_sc as plsc

VOCAB = 1000000
D = 64
SEQ = 200
N_CORES = 2
N_SUBCORES = 16
N_WORKERS = N_CORES * N_SUBCORES

NBUF = 4

TBLK = 16384
BBLK = 512
LBLK = 8


def _transpose_table(emb_t):
    def body(in_ref, out_ref):
        x = in_ref[...]
        t = jax.lax.dot_general(
            x, jnp.eye(D, dtype=jnp.float32),
            dimension_numbers=(((0,), (0,)), ((), ())),
            preferred_element_type=jnp.float32,
        )
        out_ref[:, :D] = t

    grid = (pl.cdiv(VOCAB, TBLK),)
    return pl.pallas_call(
        body,
        grid=grid,
        in_specs=[pl.BlockSpec((D, TBLK), lambda i: (0, i))],
        out_specs=pl.BlockSpec((TBLK, 2 * D), lambda i: (i, 0)),
        out_shape=jax.ShapeDtypeStruct((VOCAB, 2 * D), jnp.float32),
    )(emb_t)


@functools.lru_cache(maxsize=None)
def _make_gather(n_batch: int, l_len: int):
    per_w = n_batch // N_WORKERS
    assert n_batch % N_WORKERS == 0
    assert per_w % NBUF == 0 and per_w >= 2 * NBUF
    mesh = plsc.VectorSubcoreMesh(core_axis_name="c", subcore_axis_name="s")

    @functools.partial(
        pl.kernel,
        mesh=mesh,
        out_type=jax.ShapeDtypeStruct((n_batch, l_len, 2 * D), jnp.float32),
        compiler_params=pltpu.CompilerParams(use_tc_tiling_on_sc=True),
        scratch_types=[pltpu.VMEM((l_len,), jnp.int32) for _ in range(NBUF)]
        + [pltpu.VMEM((l_len, 2 * D), jnp.float32) for _ in range(NBUF)]
        + [pltpu.SemaphoreType.DMA for _ in range(3 * NBUF)],
    )
    def gather_kernel(x1_hbm, emb_hbm, out_hbm, *bufs):
        idx_v = list(bufs[0:NBUF])
        rows_v = list(bufs[NBUF:2 * NBUF])
        lsem = list(bufs[2 * NBUF:3 * NBUF])
        gsem = list(bufs[3 * NBUF:4 * NBUF])
        osem = list(bufs[4 * NBUF:5 * NBUF])

        wid = lax.axis_index("s") * N_CORES + lax.axis_index("c")
        b0 = wid * per_w

        def start_load(c, b):
            pltpu.async_copy(x1_hbm.at[b0 + c], idx_v[b], lsem[b])

        def wait_load(b):
            pltpu.make_async_copy(x1_hbm.at[0], idx_v[b], lsem[b]).wait()

        def start_gather(b):
            pltpu.async_copy(emb_hbm.at[idx_v[b]], rows_v[b], gsem[b])

        def wait_gather(b):
            pltpu.make_async_copy(
                emb_hbm.at[idx_v[b]], rows_v[b], gsem[b]).wait()

        def start_store(c, b):
            pltpu.async_copy(rows_v[b], out_hbm.at[b0 + c], osem[b])

        def wait_store(b):
            pltpu.make_async_copy(rows_v[b], out_hbm.at[0], osem[b]).wait()

        for b in range(NBUF):
            start_load(b, b)
        wait_load(0)
        start_gather(0)

        def steady(gg, carry):
            for b in range(NBUF):
                c = gg * NBUF + b
                bn = (b + 1) % NBUF
                @pl.when(c + 1 < per_w)
                def _():
                    wait_load(bn)

                    @pl.when(c + 1 >= NBUF)
                    def _():
                        wait_store(bn)

                    start_gather(bn)

                wait_gather(b)
                start_store(c, b)

                @pl.when(c + NBUF < per_w)
                def _():
                    start_load(c + NBUF, b)

            return carry

        lax.fori_loop(0, per_w // NBUF, steady, 0)
        for b in range(NBUF):
            wait_store(b)

    return gather_kernel


def _finalize_part(gath, x2t_p, pos_p, w2, b2, n_batch, l_off, l_len, prev):
    def body(*refs):
        g_ref, x2_ref, pos_ref, w_ref, b_ref, out_ref = refs[-6:]
        x2b = x2_ref[...]
        w = w_ref[0]
        bias = b_ref[0]
        pp = pos_ref[...] + bias[None, :]
        g2 = g_ref[...].reshape(BBLK, LBLK * 2 * D)
        t = jnp.transpose(g2)
        for l in range(LBLK):
            t_l = t[l * 2 * D:l * 2 * D + D]
            out_ref[l] = t_l + x2b[l][None, :] * w[:, None] + pp[l][:, None]

    i0 = l_off // LBLK
    grid = (l_len // LBLK, n_batch // BBLK)
    in_specs = [
        pl.BlockSpec((BBLK, LBLK, 2 * D), lambda i, j: (j, i, 0)),
        pl.BlockSpec((LBLK, BBLK), lambda i, j: (i, j)),
        pl.BlockSpec((LBLK, D), lambda i, j: (i, 0)),
        pl.BlockSpec((1, D), lambda i, j: (0, 0)),
        pl.BlockSpec((1, D), lambda i, j: (0, 0)),
    ]
    args = (gath, x2t_p, pos_p, w2, b2)
    aliases = {}
    if prev is not None:
        in_specs = [pl.BlockSpec(memory_space=pl.ANY)] + in_specs
        args = (prev,) + args
        aliases = {0: 0}
    return pl.pallas_call(
        body,
        grid=grid,
        in_specs=in_specs,
        out_specs=pl.BlockSpec((LBLK, D, BBLK), lambda i, j: (i + i0, 0, j)),
        out_shape=jax.ShapeDtypeStruct((SEQ, D, n_batch), jnp.float32),
        input_output_aliases=aliases,
    )(*args)


def kernel(x1, x2, emb_table, pos_enc, mas_w, mas_b):
    n_batch, l = x1.shape
    emb_rm = _transpose_table(jnp.transpose(emb_table))
    x2t = jnp.transpose(x2)
    w2 = mas_w.reshape(1, D)
    b2 = mas_b.reshape(1, D)
    parts = ((0, 56), (56, 56), (112, 56), (168, 32))
    gaths = [
        _make_gather(n_batch, ln)(
            lax.slice(x1, (0, off), (n_batch, off + ln)), emb_rm)
        for off, ln in parts
    ]
    out_t = None
    for (off, ln), g in zip(parts, gaths):
        out_t = _finalize_part(
            g,
            lax.slice(x2t, (off, 0), (off + ln, n_batch)),
            lax.slice(pos_enc, (off, 0), (off + ln, D)),
            w2, b2, n_batch, off, ln, out_t,
        )
    return jnp.transpose(out_t, (2, 0, 1))
